# baseline (speedup 1.0000x reference)
import functools

import jax
import jax.numpy as jnp
from jax import lax
from jax.experimental import pallas as pl
from jax.experimental.pallas import tpu as pltpu
from jax.experimental.pallas import tpu_sc as plsc

B = 16384
D = 32
NGENRE = 1000
NC, NS, L = 2, 16, 16
NW = NC * NS
B_PER_W = B // NW
CHUNK = 128
NCHUNK = B_PER_W // CHUNK
GROUPS = B_PER_W // L


def _body(vid_hbm, gid_hbm, vtab_hbm, gtab_hbm, wv_hbm, bv_hbm, out_hbm,
          vidx, gidx, vdst, gtab_v, wv, bv, outs, sem_v, sem_g):
    wid = lax.axis_index("s") * NC + lax.axis_index("c")
    base = wid * B_PER_W

    gt_copy = pltpu.async_copy(gtab_hbm, gtab_v, sem_g)
    pltpu.sync_copy(vid_hbm.at[pl.ds(base, B_PER_W)], vidx)
    pltpu.sync_copy(gid_hbm.at[pl.ds(base, B_PER_W)], gidx)
    pltpu.sync_copy(wv_hbm, wv)
    pltpu.sync_copy(bv_hbm, bv)

    def issue_body(d, carry):
        for j in range(NCHUNK):
            pltpu.async_copy(
                vtab_hbm.at[d].at[vidx.at[pl.ds(j * CHUNK, CHUNK)]],
                vdst.at[d].at[pl.ds(j * CHUNK, CHUNK)],
                sem_v)
        return carry

    lax.fori_loop(0, D, issue_body, 0)
    gt_copy.wait()
    pltpu.make_async_copy(
        vtab_hbm.at[:, pl.ds(0, B_PER_W)], vdst, sem_v).wait()

    lanes = lax.iota(jnp.int32, L)
    w = wv[...]
    bb = bv[...]

    def group_body(g, carry):
        gvec = gidx[pl.ds(g * L, L)]
        dot = jnp.zeros((L,), jnp.float32)
        mm = jnp.zeros((L,), jnp.float32)
        gg = jnp.zeros((L,), jnp.float32)
        for d in range(D):
            m = vdst[d, pl.ds(g * L, L)]
            ge = plsc.load_gather(gtab_v, [jnp.full((L,), d, jnp.int32), gvec])
            dot = dot + m * ge
            mm = mm + m * m
            gg = gg + ge * ge
        x = jnp.maximum(mm, 1e-12) * jnp.maximum(gg, 1e-12)
        i = plsc.bitcast(x, jnp.int32)
        y = plsc.bitcast(jnp.int32(0x5F3759DF) - (i >> 1), jnp.float32)
        for _ in range(3):
            y = y * (1.5 - 0.5 * x * y * y)
        logit = dot * y * w + bb
        prob = 1.0 / (1.0 + jnp.exp(-logit))
        outs[pl.ds(g * L, L)] = prob
        return carry

    lax.fori_loop(0, GROUPS, group_body, 0)
    pltpu.sync_copy(outs, out_hbm.at[pl.ds(base, B_PER_W)])


@jax.jit
def _run(vid, gid, vtab_t, gtab_t, wv, bv):
    mesh = plsc.VectorSubcoreMesh(
        core_axis_name="c", subcore_axis_name="s",
        num_cores=NC, num_subcores=NS)
    f = functools.partial(
        pl.kernel,
        out_type=jax.ShapeDtypeStruct((B,), jnp.float32),
        mesh=mesh,
        compiler_params=pltpu.CompilerParams(
            needs_layout_passes=False, use_tc_tiling_on_sc=False),
        scratch_types=[
            pltpu.VMEM((B_PER_W,), jnp.int32),
            pltpu.VMEM((B_PER_W,), jnp.int32),
            pltpu.VMEM((D, B_PER_W), jnp.float32),
            pltpu.VMEM((D, NGENRE), jnp.float32),
            pltpu.VMEM((L,), jnp.float32),
            pltpu.VMEM((L,), jnp.float32),
            pltpu.VMEM((B_PER_W,), jnp.float32),
            pltpu.SemaphoreType.DMA,
            pltpu.SemaphoreType.DMA,
        ],
    )(_body)
    return f(vid, gid, vtab_t, gtab_t, wv, bv)


def kernel(video_ids, genre_ids, video_table, genre_table, W, b):
    vid = video_ids.astype(jnp.int32)
    gid = genre_ids.astype(jnp.int32)
    wv = jnp.full((L,), W[0, 0], dtype=jnp.float32)
    bv = jnp.full((L,), b[0], dtype=jnp.float32)
    out = _run(vid, gid, video_table.T, genre_table.T, wv, bv)
    return out.reshape(B, 1)


# speedup vs baseline: 7.8416x; 7.8416x over previous
import functools

import jax
import jax.numpy as jnp
from jax import lax
from jax.experimental import pallas as pl
from jax.experimental.pallas import tpu as pltpu
from jax.experimental.pallas import tpu_sc as plsc

B = 16384
D = 32
NC, NS, L = 2, 16, 16
NW = NC * NS
B_PER_W = B // NW
GROUPS = B_PER_W // L


def _body(vid_hbm, gid_hbm, vtab_hbm, gtab_hbm, wv_hbm, bv_hbm, out_hbm,
          vidx_v, gidx_v, vdst, gdst, wv, bv, outs, sem_v, sem_g):
    wid = lax.axis_index("s") * NC + lax.axis_index("c")
    base = wid * B_PER_W

    pltpu.sync_copy(vid_hbm.at[pl.ds(base, B_PER_W)], vidx_v)
    pltpu.sync_copy(gid_hbm.at[pl.ds(base, B_PER_W)], gidx_v)
    pltpu.sync_copy(wv_hbm, wv)
    pltpu.sync_copy(bv_hbm, bv)

    def chunk_body(c, carry):
        vv = vidx_v[pl.ds(c * L, L)]
        gv = gidx_v[pl.ds(c * L, L)]
        for k in range(L):
            r = c * (L // 4) + k // 4
            col = (k % 4) * D
            pltpu.async_copy(vtab_hbm.at[vv[k]], vdst.at[r, pl.ds(col, D)],
                             sem_v)
            pltpu.async_copy(gtab_hbm.at[gv[k]], gdst.at[r, pl.ds(col, D)],
                             sem_g)
        return carry

    lax.fori_loop(0, GROUPS, chunk_body, 0)
    pltpu.make_async_copy(vtab_hbm.at[pl.ds(0, B_PER_W // 4)], vdst, sem_v).wait()
    pltpu.make_async_copy(gtab_hbm.at[pl.ds(0, B_PER_W // 4)], gdst, sem_g).wait()

    lanes = lax.iota(jnp.int32, L)
    w = wv[...]
    bb = bv[...]

    def group_body(g, carry):
        rows = g * L + lanes
        rvec = rows >> 2
        cvec0 = (rows & 3) * D
        dot = jnp.zeros((L,), jnp.float32)
        mm = jnp.zeros((L,), jnp.float32)
        gg = jnp.zeros((L,), jnp.float32)
        for d in range(D):
            m = plsc.load_gather(vdst, [rvec, cvec0 + d])
            ge = plsc.load_gather(gdst, [rvec, cvec0 + d])
            dot = dot + m * ge
            mm = mm + m * m
            gg = gg + ge * ge
        x = jnp.maximum(mm, 1e-12) * jnp.maximum(gg, 1e-12)
        i = plsc.bitcast(x, jnp.int32)
        y = plsc.bitcast(jnp.int32(0x5F3759DF) - (i >> 1), jnp.float32)
        for _ in range(3):
            y = y * (1.5 - 0.5 * x * y * y)
        logit = dot * y * w + bb
        prob = 1.0 / (1.0 + jnp.exp(-logit))
        outs[pl.ds(g * L, L)] = prob
        return carry

    lax.fori_loop(0, GROUPS, group_body, 0)
    pltpu.sync_copy(outs, out_hbm.at[pl.ds(base, B_PER_W)])


@jax.jit
def _run(vid, gid, vtab, gtab, wv, bv):
    mesh = plsc.VectorSubcoreMesh(
        core_axis_name="c", subcore_axis_name="s",
        num_cores=NC, num_subcores=NS)
    f = functools.partial(
        pl.kernel,
        out_type=jax.ShapeDtypeStruct((B,), jnp.float32),
        mesh=mesh,
        compiler_params=pltpu.CompilerParams(
            needs_layout_passes=False, use_tc_tiling_on_sc=True),
        scratch_types=[
            pltpu.VMEM((B_PER_W,), jnp.int32),
            pltpu.VMEM((B_PER_W,), jnp.int32),
            pltpu.VMEM((B_PER_W // 4, 4 * D), jnp.float32),
            pltpu.VMEM((B_PER_W // 4, 4 * D), jnp.float32),
            pltpu.VMEM((L,), jnp.float32),
            pltpu.VMEM((L,), jnp.float32),
            pltpu.VMEM((B_PER_W,), jnp.float32),
            pltpu.SemaphoreType.DMA,
            pltpu.SemaphoreType.DMA,
        ],
    )(_body)
    return f(vid, gid, vtab, gtab, wv, bv)


def kernel(video_ids, genre_ids, video_table, genre_table, W, b):
    vid = video_ids.astype(jnp.int32)
    gid = genre_ids.astype(jnp.int32)
    wv = jnp.full((L,), W[0, 0], dtype=jnp.float32)
    bv = jnp.full((L,), b[0], dtype=jnp.float32)
    out = _run(vid, gid, video_table, genre_table, wv, bv)
    return out.reshape(B, 1)
